# baseline (speedup 1.0000x reference)
import jax
import jax.numpy as jnp
from jax import lax
from jax.experimental import pallas as pl
from jax.experimental.pallas import tpu as pltpu

NSTREAM = 16


def _body(x_hbm, p_ref, v_ref, *scratch):
    faces = scratch[:NSTREAM]
    sems = scratch[NSTREAM:]
    B = v_ref.shape[0]
    chunk = B // NSTREAM
    copies = []
    for k in range(NSTREAM):
        cp = pltpu.make_async_copy(
            x_hbm.at[pl.ds(k * chunk, chunk), 0, 0],
            faces[k],
            sems[k],
        )
        cp.start()
        copies.append(cp)
    for cp in copies:
        cp.wait()
    any_even = False
    any_odd = False
    for k in range(NSTREAM):
        step_k = faces[k][:, 0:1]
        v_ref[pl.ds(k * chunk, chunk), :] = 100.0 - step_k
        par_k = jnp.bitwise_and(step_k.astype(jnp.int32), 1)
        any_odd = jnp.logical_or(any_odd, jnp.max(par_k) > 0)
        any_even = jnp.logical_or(any_even, jnp.min(par_k) < 1)
    c0 = jnp.where(any_even, 0.8, 0.2)
    c1 = jnp.where(any_odd, 0.8, 0.2)
    col = lax.broadcasted_iota(jnp.int32, (B, 2), 1)
    p_ref[:, :] = jnp.where(col == 0, c0, c1)


def kernel(x):
    B = x.shape[0]
    W = x.shape[3]
    chunk = B // NSTREAM
    P, v = pl.pallas_call(
        _body,
        in_specs=[pl.BlockSpec(memory_space=pl.ANY)],
        out_specs=(
            pl.BlockSpec((B, 2), lambda: (0, 0)),
            pl.BlockSpec((B, 1), lambda: (0, 0)),
        ),
        out_shape=(
            jax.ShapeDtypeStruct((B, 2), jnp.float32),
            jax.ShapeDtypeStruct((B, 1), jnp.float32),
        ),
        scratch_shapes=(
            [pltpu.VMEM((chunk, W), jnp.float32) for _ in range(NSTREAM)]
            + [pltpu.SemaphoreType.DMA for _ in range(NSTREAM)]
        ),
    )(x)
    return (P, v)
